# baseline (speedup 1.0000x reference)
import functools

import jax
import jax.numpy as jnp
from jax import lax
from jax.experimental import pallas as pl
from jax.experimental.pallas import tpu as pltpu
from jax.experimental.pallas import tpu_sc as plsc

N_NODES = 10000
D = 128
N_EDGES = 320000
NC = 2
NS = 16
NW = NC * NS
EPW = N_EDGES // NW
CHUNK = 64
EPW_PAD = 10240
NCHUNK = EPW_PAD // CHUNK
SB = 40
NSB = NCHUNK // SB
NBUF = 4
N_PAD = 10240
ROWS_PT = N_PAD // NS


def _sc_aggregate(x, src_r, dst_r, zeros):
    mesh = plsc.VectorSubcoreMesh(core_axis_name="c", subcore_axis_name="s")

    @functools.partial(
        pl.kernel,
        out_type=jax.ShapeDtypeStruct((NC, N_PAD, D), jnp.float32),
        mesh=mesh,
        scratch_types=[
            pltpu.VMEM((SB, CHUNK), jnp.int32),
            pltpu.VMEM((SB, CHUNK), jnp.int32),
            pltpu.VMEM((NBUF, CHUNK, D), jnp.float32),
            pltpu.VMEM_SHARED((N_PAD, D), jnp.float32),
            pltpu.SemaphoreType.DMA,
            pltpu.SemaphoreType.DMA,
            pltpu.SemaphoreType.DMA,
            pltpu.SemaphoreType.DMA,
        ],
    )
    def agg_kernel(x_hbm, src_hbm, dst_hbm, z_hbm, out_hbm,
                   src_v, dst_v, rows_v, acc, s0, s1, s2, s3):
        c = lax.axis_index("c")
        s = lax.axis_index("s")
        wid = s * NC + c
        r0 = s * ROWS_PT
        pltpu.sync_copy(z_hbm.at[pl.ds(r0, ROWS_PT)], acc.at[pl.ds(r0, ROWS_PT)])
        plsc.subcore_barrier()

        sems = (s0, s1, s2, s3)

        def g_wait(k):
            pltpu.make_async_copy(x_hbm.at[src_v.at[0]], rows_v.at[k],
                                  sems[k]).wait()

        def s_wait(k):
            pltpu.make_async_copy(rows_v.at[k], acc.at[dst_v.at[0]],
                                  sems[k]).wait()

        for sb in range(NSB):
            pltpu.sync_copy(src_hbm.at[wid, sb], src_v)
            pltpu.sync_copy(dst_hbm.at[wid, sb], dst_v)
            for k in range(NBUF):
                pltpu.async_copy(x_hbm.at[src_v.at[k]], rows_v.at[k], sems[k])

            def body(g, carry):
                j0 = NBUF * g
                for k in range(NBUF):
                    g_wait(k)
                    pltpu.async_copy(rows_v.at[k], acc.at[dst_v.at[j0 + k]],
                                     sems[k], add=True)

                @pl.when(g < SB // NBUF - 1)
                def _():
                    for k in range(NBUF):
                        s_wait(k)
                        pltpu.async_copy(x_hbm.at[src_v.at[j0 + NBUF + k]],
                                         rows_v.at[k], sems[k])

                return carry

            lax.fori_loop(0, SB // NBUF, body, 0)
            for k in range(NBUF):
                s_wait(k)

        plsc.subcore_barrier()
        pltpu.sync_copy(acc.at[pl.ds(r0, ROWS_PT)],
                        out_hbm.at[c, pl.ds(r0, ROWS_PT)])

    return agg_kernel(x, src_r, dst_r, zeros)


def _tc_linear_relu(p, W, b2):
    BM = 1000

    def body(p_ref, w_ref, b_ref, o_ref):
        a = p_ref[0] + p_ref[1]
        y = lax.dot_general(a, w_ref[...], (((1,), (1,)), ((), ())),
                            preferred_element_type=jnp.float32)
        o_ref[...] = jnp.maximum(y + b_ref[...], 0.0)

    return pl.pallas_call(
        body,
        grid=(N_NODES // BM,),
        in_specs=[
            pl.BlockSpec((NC, BM, D), lambda i: (0, i, 0)),
            pl.BlockSpec((D, D), lambda i: (0, 0)),
            pl.BlockSpec((1, D), lambda i: (0, 0)),
        ],
        out_specs=pl.BlockSpec((BM, D), lambda i: (i, 0)),
        out_shape=jax.ShapeDtypeStruct((N_NODES, D), jnp.float32),
    )(p, W, b2)


def kernel(x, edge_index, W, b):
    pad = EPW_PAD - EPW
    src = jnp.pad(edge_index[0].astype(jnp.int32).reshape(NW, EPW),
                  ((0, 0), (0, pad))).reshape(NW, NSB, SB, CHUNK)
    dst = jnp.pad(edge_index[1].astype(jnp.int32).reshape(NW, EPW),
                  ((0, 0), (0, pad)),
                  constant_values=N_NODES).reshape(NW, NSB, SB, CHUNK)
    zeros = jnp.zeros((N_PAD, D), jnp.float32)
    p = _sc_aggregate(x, src, dst, zeros)
    return _tc_linear_relu(p, W, b.reshape(1, D))
